# Initial kernel scaffold; baseline (speedup 1.0000x reference)
import jax
import jax.numpy as jnp
from jax.experimental import pallas as pl


def kernel(x, router_w, gating_w, linear_w, per_expert_scale, router_scale):
    raise NotImplementedError("write your pallas kernel here")


# speedup vs baseline: 3.9504x; 3.9504x over previous
import functools

import jax
import jax.numpy as jnp
from jax import lax
from jax.experimental import pallas as pl
from jax.experimental.pallas import tpu as pltpu

F = 1024
H = 4096
NE = 8
TOPK = 2
NTOK = 4096
NP = NTOK * TOPK
BM = 256
NBLK = NP // BM
NSTEPS = NBLK + NE - 1
TB = 128
NRB = NTOK // TB
_RSQRT_F = float(1.0 / (F ** 0.5))


def _router_kernel(x_ref, rw_ref, rs_ref, pes_ref,
                   e_ref, r_ref, w_ref, cnt_ref, acc_ref):
    i = pl.program_id(0)

    @pl.when(i == 0)
    def _init():
        acc_ref[...] = jnp.zeros_like(acc_ref)

    xb = x_ref[...]
    var = jnp.mean(xb * xb, axis=-1, keepdims=True)
    xn = xb * lax.rsqrt(var + 1e-6)
    ri = xn * _RSQRT_F * rs_ref[...]
    logits = jnp.dot(ri, rw_ref[...],
                     preferred_element_type=jnp.float32)

    m = jnp.max(logits, axis=-1, keepdims=True)
    p = jnp.exp(logits - m)
    p = p / jnp.sum(p, axis=-1, keepdims=True)

    lane = lax.broadcasted_iota(jnp.int32, (TB, NE), 1)
    big = jnp.int32(NE + 1)
    m1 = jnp.max(logits, axis=-1, keepdims=True)
    i1 = jnp.min(jnp.where(logits == m1, lane, big), axis=-1, keepdims=True)
    l2 = jnp.where(lane == i1, -jnp.inf, logits)
    m2 = jnp.max(l2, axis=-1, keepdims=True)
    i2 = jnp.min(jnp.where(l2 == m2, lane, big), axis=-1, keepdims=True)

    oh1 = (lane == i1).astype(jnp.float32)
    oh2 = (lane == i2).astype(jnp.float32)
    p1 = jnp.sum(oh1 * p, axis=-1, keepdims=True)
    p2 = jnp.sum(oh2 * p, axis=-1, keepdims=True)
    rf = p1 + p2
    rf = jnp.where(rf > 0.0, rf, 1.0)
    pes = pes_ref[...]
    s1 = jnp.sum(oh1 * pes, axis=-1, keepdims=True)
    s2 = jnp.sum(oh2 * pes, axis=-1, keepdims=True)
    w1 = p1 / rf * s1
    w2 = p2 / rf * s2

    ohs = jnp.concatenate([oh1, oh2], axis=0)
    n2 = 2 * TB
    row = lax.broadcasted_iota(jnp.int32, (n2, n2), 0)
    col = lax.broadcasted_iota(jnp.int32, (n2, n2), 1)
    tri = (col < row).astype(jnp.float32)
    strict = jnp.dot(tri, ohs, preferred_element_type=jnp.float32)
    prev = acc_ref[0:1, 0:NE]
    rank = jnp.sum((strict + prev) * ohs, axis=-1, keepdims=True)
    acc_ref[0:1, 0:NE] = prev + jnp.sum(ohs, axis=0, keepdims=True)

    e_ref[...] = jnp.concatenate([i1, i2], axis=1)
    r_ref[...] = jnp.concatenate([rank[:TB], rank[TB:]],
                                 axis=1).astype(jnp.int32)
    w_ref[...] = jnp.concatenate([w1, w2], axis=1)
    cnt_ref[...] = acc_ref[0:1, 0:NE].astype(jnp.int32)


def _router(xf, router_w, router_scale, per_expert_scale):
    return pl.pallas_call(
        _router_kernel,
        grid=(NRB,),
        in_specs=[
            pl.BlockSpec((TB, F), lambda i: (i, 0)),
            pl.BlockSpec((F, NE), lambda i: (0, 0)),
            pl.BlockSpec((1, F), lambda i: (0, 0)),
            pl.BlockSpec((1, NE), lambda i: (0, 0)),
        ],
        out_specs=[
            pl.BlockSpec((TB, TOPK), lambda i: (i, 0)),
            pl.BlockSpec((TB, TOPK), lambda i: (i, 0)),
            pl.BlockSpec((TB, TOPK), lambda i: (i, 0)),
            pl.BlockSpec((1, NE), lambda i: (0, 0)),
        ],
        out_shape=[
            jax.ShapeDtypeStruct((NTOK, TOPK), jnp.int32),
            jax.ShapeDtypeStruct((NTOK, TOPK), jnp.int32),
            jax.ShapeDtypeStruct((NTOK, TOPK), jnp.float32),
            jax.ShapeDtypeStruct((1, NE), jnp.int32),
        ],
        scratch_shapes=[pltpu.VMEM((8, 128), jnp.float32)],
    )(xf, router_w, router_scale.reshape(1, F),
      per_expert_scale.reshape(1, NE))


def _group_metadata(counts):
    counts = counts.astype(jnp.int32)
    ends = jnp.cumsum(counts)
    starts = ends - counts
    fb = starts // BM
    lb = jnp.where(counts > 0, (ends - 1) // BM, -1)
    tiles = jnp.where(counts > 0, lb - fb + 1, 0)
    tiles_cum = jnp.cumsum(tiles)
    total = tiles_cum[-1]
    step = jnp.arange(NSTEPS, dtype=jnp.int32)
    e_s = jnp.searchsorted(tiles_cum, step, side='right').astype(jnp.int32)
    e_c = jnp.clip(e_s, 0, NE - 1)
    prev_tiles = jnp.where(e_c > 0, tiles_cum[e_c - 1], 0)
    b_s = fb[e_c] + (step - prev_tiles)
    b_s = jnp.clip(b_s, 0, NBLK - 1)
    lo = jnp.maximum(starts[e_c], b_s * BM)
    hi = jnp.minimum(ends[e_c], (b_s + 1) * BM)
    hi = jnp.where(step < total, hi, lo)
    return e_s * 0 + e_c, b_s.astype(jnp.int32), lo.astype(jnp.int32), hi.astype(jnp.int32)


def _row_mask(b, lo, hi):
    rows = b * BM + lax.broadcasted_iota(jnp.int32, (BM, 1), 0)
    return (rows >= lo) & (rows < hi)


def _is_first(i, b, b_ref):
    bp = b_ref[jnp.maximum(i - 1, 0)]
    return jnp.logical_or(i == 0, b != bp)


BH = 1024
NH = H // BH


def _gmm1_kernel(e_ref, b_ref, lo_ref, hi_ref, x_ref, g_ref, act_ref):
    i = pl.program_id(1)
    b, lo, hi = b_ref[i], lo_ref[i], hi_ref[i]
    mask = _row_mask(b, lo, hi)
    xb = jnp.where(mask, x_ref[...], 0.0)
    dn = (((1,), (1,)), ((), ()))
    x1 = lax.dot_general(xb, g_ref[0, 0], dn,
                         preferred_element_type=jnp.float32)
    x2 = lax.dot_general(xb, g_ref[0, 1], dn,
                         preferred_element_type=jnp.float32)
    act = jax.nn.gelu(x1) * x2
    first = _is_first(i, b, b_ref)

    @pl.when(first)
    def _set():
        act_ref[...] = act

    @pl.when(jnp.logical_not(first))
    def _add():
        act_ref[...] += act


def _gmm2_kernel(e_ref, b_ref, lo_ref, hi_ref, a_ref, lw_ref, out_ref):
    i = pl.program_id(0)
    b, lo, hi = b_ref[i], lo_ref[i], hi_ref[i]
    mask = _row_mask(b, lo, hi)
    ab = jnp.where(mask, a_ref[...], 0.0)
    y = lax.dot_general(ab, lw_ref[0], (((1,), (0,)), ((), ())),
                        preferred_element_type=jnp.float32)
    first = _is_first(i, b, b_ref)

    @pl.when(first)
    def _set():
        out_ref[...] = y

    @pl.when(jnp.logical_not(first))
    def _add():
        out_ref[...] += y


def _gmm1(meta, sorted_xs, gating_w):
    grid_spec = pltpu.PrefetchScalarGridSpec(
        num_scalar_prefetch=4,
        grid=(NH, NSTEPS),
        in_specs=[
            pl.BlockSpec((BM, F), lambda h, i, e, b, lo, hi: (b[i], 0)),
            pl.BlockSpec((1, 2, BH, F),
                         lambda h, i, e, b, lo, hi: (e[i], 0, h, 0)),
        ],
        out_specs=pl.BlockSpec((BM, BH),
                               lambda h, i, e, b, lo, hi: (b[i], h)),
        scratch_shapes=[],
    )
    return pl.pallas_call(
        _gmm1_kernel,
        grid_spec=grid_spec,
        out_shape=jax.ShapeDtypeStruct((NP, H), jnp.float32),
    )(*meta, sorted_xs, gating_w)


def _gmm2(meta, act, linear_w):
    grid_spec = pltpu.PrefetchScalarGridSpec(
        num_scalar_prefetch=4,
        grid=(NSTEPS,),
        in_specs=[
            pl.BlockSpec((BM, H), lambda i, e, b, lo, hi: (b[i], 0)),
            pl.BlockSpec((1, H, F), lambda i, e, b, lo, hi: (e[i], 0, 0)),
        ],
        out_specs=pl.BlockSpec((BM, F), lambda i, e, b, lo, hi: (b[i], 0)),
        scratch_shapes=[],
    )
    return pl.pallas_call(
        _gmm2_kernel,
        grid_spec=grid_spec,
        out_shape=jax.ShapeDtypeStruct((NP, F), jnp.float32),
    )(*meta, act, linear_w)


def kernel(x, router_w, gating_w, linear_w, per_expert_scale, router_scale):
    g, s, f = x.shape
    xf = x.reshape(-1, f)

    e, r, w, counts = _router(xf, router_w, router_scale, per_expert_scale)
    counts = counts[0]
    ends = jnp.cumsum(counts)
    starts = ends - counts
    pos = starts[e] + r

    meta = _group_metadata(counts)

    sorted_xs = (jnp.zeros((NP, f), x.dtype)
                 .at[pos[:, 0]].set(xf)
                 .at[pos[:, 1]].set(xf))

    act = _gmm1(meta, sorted_xs, gating_w)
    eo = _gmm2(meta, act, linear_w)

    gathered = eo[pos]
    out = jnp.einsum('tkd,tk->td', gathered, w,
                     preferred_element_type=jnp.float32)
    return out.reshape(g, s, f)
